# Initial kernel scaffold; baseline (speedup 1.0000x reference)
import jax
import jax.numpy as jnp
from jax.experimental import pallas as pl


def kernel(source_nodes, destination_nodes, negative_nodes, edge_times, edge_idxs, node_features, edge_features, nbr_ids, nbr_eidx, nbr_ts, time_w, time_b, W1, b1, W2, b2, fc1_w, fc1_b, fc2_w, fc2_b, dec_w, dec_b):
    raise NotImplementedError("write your pallas kernel here")


# speedup vs baseline: 1.9439x; 1.9439x over previous
import functools

import jax
import jax.numpy as jnp
from jax import lax
from jax.experimental import pallas as pl
from jax.experimental.pallas import tpu as pltpu

D_FEAT = 128
D_TIME = 128
D_EDGE = 16
K_NBR = 20
LVL1_ROWS = 800


def _layer1_body(a1_ref, a2_ref, nts_ref, nf1_ref, t1_ref, tw_ref, tb_ref,
                 w1a_ref, w1b_ref, w1c_ref, b1_ref,
                 w2a_ref, w2b_ref, w2c_ref, b2_ref, out_ref):
    a1 = a1_ref[...]
    nts = nts_ref[...]
    t1 = t1_ref[...]
    tw = tw_ref[...]
    tb = tb_ref[...]
    csum = jnp.zeros((a1.shape[0], D_TIME), jnp.float32)
    for k in range(K_NBR):
        csum = csum + jnp.cos((t1 - nts[:, k:k + 1]) * tw + tb)
    s = (jnp.dot(a1, w1a_ref[...], preferred_element_type=jnp.float32)
         + jnp.dot(csum, w1b_ref[...], preferred_element_type=jnp.float32)
         + jnp.dot(a2_ref[...], w1c_ref[...], preferred_element_type=jnp.float32)
         + K_NBR * b1_ref[...])
    s = jnp.maximum(s, 0.0)
    srcte = jnp.cos(tb)
    h = (jnp.dot(s, w2a_ref[...], preferred_element_type=jnp.float32)
         + jnp.dot(nf1_ref[...], w2b_ref[...], preferred_element_type=jnp.float32)
         + (jnp.dot(srcte, w2c_ref[...], preferred_element_type=jnp.float32)
            + b2_ref[...]))
    out_ref[...] = h.reshape(-1, K_NBR, D_FEAT).sum(axis=1)


def _layer2_body(b1s_ref, a2b_ref, nts_ref, nf2_ref, t2_ref, tw_ref, tb_ref,
                 w1a_ref, w1b_ref, w1c_ref, b1_ref,
                 w2a_ref, w2b_ref, w2c_ref, b2_ref,
                 fc1w_ref, fc1b_ref, fc2w_ref, fc2b_ref,
                 decw_ref, decb_ref,
                 pos_ref, neg_ref, dec_ref):
    nts = nts_ref[...]
    t2 = t2_ref[...]
    tw = tw_ref[...]
    tb = tb_ref[...]
    csum = jnp.zeros((600, D_TIME), jnp.float32)
    for k in range(K_NBR):
        csum = csum + jnp.cos((t2 - nts[:, k:k + 1]) * tw + tb)
    s = (jnp.dot(b1s_ref[...], w1a_ref[...], preferred_element_type=jnp.float32)
         + jnp.dot(csum, w1b_ref[...], preferred_element_type=jnp.float32)
         + jnp.dot(a2b_ref[...], w1c_ref[...], preferred_element_type=jnp.float32)
         + K_NBR * b1_ref[...])
    s = jnp.maximum(s, 0.0)
    srcte = jnp.cos(tb)
    emb = (jnp.dot(s, w2a_ref[...], preferred_element_type=jnp.float32)
           + jnp.dot(nf2_ref[...], w2b_ref[...], preferred_element_type=jnp.float32)
           + (jnp.dot(srcte, w2c_ref[...], preferred_element_type=jnp.float32)
              + b2_ref[...]))
    src_e = emb[:200]
    dst_e = emb[200:400]
    neg_e = emb[400:600]

    def merge(a, b):
        hh = (jnp.dot(a, fc1w_ref[:D_FEAT], preferred_element_type=jnp.float32)
              + jnp.dot(b, fc1w_ref[D_FEAT:], preferred_element_type=jnp.float32)
              + fc1b_ref[...])
        hh = jnp.maximum(hh, 0.0)
        return jnp.dot(hh, fc2w_ref[...], preferred_element_type=jnp.float32) + fc2b_ref[...]

    pos_ref[...] = jax.nn.sigmoid(merge(src_e, dst_e))
    neg_ref[...] = jax.nn.sigmoid(merge(src_e, neg_e))
    dec_ref[...] = (jnp.dot(emb[:400], decw_ref[...], preferred_element_type=jnp.float32)
                    + decb_ref[...])


def _full(a):
    return pl.BlockSpec(a.shape, lambda *_: (0,) * a.ndim)


def kernel(source_nodes, destination_nodes, negative_nodes, edge_times, edge_idxs,
           node_features, edge_features, nbr_ids, nbr_eidx, nbr_ts,
           time_w, time_b, W1, b1, W2, b2, fc1_w, fc1_b, fc2_w, fc2_b, dec_w, dec_b):
    n = source_nodes.shape[0]
    M = 3 * n
    n2 = jnp.concatenate([source_nodes, destination_nodes, negative_nodes])
    t2 = jnp.concatenate([edge_times, edge_times, edge_times])

    NBR2 = jnp.take(nbr_ids, n2, axis=0)
    EIX2 = jnp.take(nbr_eidx, n2, axis=0)
    NTS2 = jnp.take(nbr_ts, n2, axis=0)
    NF2 = jnp.take(node_features, n2, axis=0)
    n1 = NBR2.reshape(-1)
    NBR1 = jnp.take(nbr_ids, n1, axis=0)
    EIX1 = jnp.take(nbr_eidx, n1, axis=0)
    NTS1 = jnp.take(nbr_ts, n1, axis=0)
    NF1 = jnp.take(node_features, n1, axis=0)
    A1 = jnp.take(node_features, NBR1.reshape(-1), axis=0).reshape(-1, K_NBR, D_FEAT).sum(axis=1)
    A2 = jnp.take(edge_features, EIX1.reshape(-1), axis=0).reshape(-1, K_NBR, D_EDGE).sum(axis=1)
    A2b = jnp.take(edge_features, EIX2.reshape(-1), axis=0).reshape(-1, K_NBR, D_EDGE).sum(axis=1)

    tw2 = time_w.reshape(1, D_TIME)
    tb2 = time_b.reshape(1, D_TIME)
    T1 = jnp.repeat(t2, K_NBR).reshape(-1, 1)
    W1a0, W1b0, W1c0 = W1[0, :D_FEAT], W1[0, D_FEAT:D_FEAT + D_TIME], W1[0, D_FEAT + D_TIME:]
    W2a0, W2b0, W2c0 = W2[0, :D_FEAT], W2[0, D_FEAT:2 * D_FEAT], W2[0, 2 * D_FEAT:]
    W1a1, W1b1, W1c1 = W1[1, :D_FEAT], W1[1, D_FEAT:D_FEAT + D_TIME], W1[1, D_FEAT + D_TIME:]
    W2a1, W2b1, W2c1 = W2[1, :D_FEAT], W2[1, D_FEAT:2 * D_FEAT], W2[1, 2 * D_FEAT:]

    n_lvl1 = M * K_NBR
    grid = n_lvl1 // LVL1_ROWS
    row = lambda i: (i, 0)
    B1s = pl.pallas_call(
        _layer1_body,
        grid=(grid,),
        in_specs=[
            pl.BlockSpec((LVL1_ROWS, D_FEAT), row),
            pl.BlockSpec((LVL1_ROWS, D_EDGE), row),
            pl.BlockSpec((LVL1_ROWS, K_NBR), row),
            pl.BlockSpec((LVL1_ROWS, D_FEAT), row),
            pl.BlockSpec((LVL1_ROWS, 1), row),
            _full(tw2), _full(tb2),
            _full(W1a0), _full(W1b0), _full(W1c0), _full(b1[0:1]),
            _full(W2a0), _full(W2b0), _full(W2c0), _full(b2[0:1]),
        ],
        out_specs=pl.BlockSpec((LVL1_ROWS // K_NBR, D_FEAT), row),
        out_shape=jax.ShapeDtypeStruct((M, D_FEAT), jnp.float32),
    )(A1, A2, NTS1, NF1, T1, tw2, tb2,
      W1a0, W1b0, W1c0, b1[0:1], W2a0, W2b0, W2c0, b2[0:1])

    pos, neg, dec = pl.pallas_call(
        _layer2_body,
        in_specs=[_full(B1s), _full(A2b), _full(NTS2), _full(NF2),
                  pl.BlockSpec((M, 1), lambda *_: (0, 0)),
                  _full(tw2), _full(tb2),
                  _full(W1a1), _full(W1b1), _full(W1c1), _full(b1[1:2]),
                  _full(W2a1), _full(W2b1), _full(W2c1), _full(b2[1:2]),
                  _full(fc1_w), _full(fc1_b.reshape(1, D_FEAT)),
                  _full(fc2_w), _full(fc2_b.reshape(1, 1)),
                  _full(dec_w), _full(dec_b.reshape(1, D_FEAT))],
        out_specs=[pl.BlockSpec((n, 1), lambda *_: (0, 0)),
                   pl.BlockSpec((n, 1), lambda *_: (0, 0)),
                   pl.BlockSpec((2 * n, D_FEAT), lambda *_: (0, 0))],
        out_shape=[jax.ShapeDtypeStruct((n, 1), jnp.float32),
                   jax.ShapeDtypeStruct((n, 1), jnp.float32),
                   jax.ShapeDtypeStruct((2 * n, D_FEAT), jnp.float32)],
    )(B1s, A2b, NTS2, NF2, t2.reshape(M, 1), tw2, tb2,
      W1a1, W1b1, W1c1, b1[1:2], W2a1, W2b1, W2c1, b2[1:2],
      fc1_w, fc1_b.reshape(1, D_FEAT), fc2_w, fc2_b.reshape(1, 1),
      dec_w, dec_b.reshape(1, D_FEAT))

    return pos.reshape(-1), neg.reshape(-1), dec, NF2[:2 * n]


# speedup vs baseline: 3.3457x; 1.7211x over previous
import functools

import jax
import jax.numpy as jnp
from jax import lax
from jax.experimental import pallas as pl
from jax.experimental.pallas import tpu as pltpu
from jax.experimental.pallas import tpu_sc as plsc

D_FEAT = 128
D_TIME = 128
D_EDGE = 16
K_NBR = 20
LVL1_ROWS = 800

_NC = 2
_NS = 16
_NW = _NC * _NS


def _sc_mesh():
    return plsc.VectorSubcoreMesh(core_axis_name="c", subcore_axis_name="s")


_SC_PARAMS = pltpu.CompilerParams(use_tc_tiling_on_sc=False)


def _wid():
    return lax.axis_index("s") * _NC + lax.axis_index("c")


def _make_row_gather(n_pad, chunk):
    pw = n_pad // _NW
    assert pw % chunk == 0 and chunk <= 128

    def body(idx_hbm, pk_hbm, nf_hbm, opk_hbm, onf_hbm, idx_v, pk_v, nf_v, sem):
        base = _wid() * pw
        pltpu.sync_copy(idx_hbm.at[pl.ds(base, pw)], idx_v)
        handles = []
        for t, b in ((pk_hbm, pk_v), (nf_hbm, nf_v)):
            for c in range(pw // chunk):
                handles.append(pltpu.async_copy(
                    t.at[idx_v.at[pl.ds(c * chunk, chunk)]],
                    b.at[pl.ds(c * chunk, chunk)], sem))
        for h in handles:
            h.wait()
        pltpu.sync_copy(pk_v, opk_hbm.at[pl.ds(base, pw)])
        pltpu.sync_copy(nf_v, onf_hbm.at[pl.ds(base, pw)])

    def run(idx, packed, nf):
        out_type = [jax.ShapeDtypeStruct((n_pad, 64), jnp.int32),
                    jax.ShapeDtypeStruct((n_pad, D_FEAT), jnp.float32)]
        scratch = [pltpu.VMEM((pw,), jnp.int32),
                   pltpu.VMEM((pw, 64), jnp.int32),
                   pltpu.VMEM((pw, D_FEAT), jnp.float32),
                   pltpu.SemaphoreType.DMA]
        return pl.kernel(body, mesh=_sc_mesh(), out_type=out_type,
                         scratch_types=scratch,
                         compiler_params=_SC_PARAMS)(idx, packed, nf)

    return run


def _make_gather_segsum(n_pad, d, cn):
    npw = n_pad // _NW
    rpw = npw * K_NBR
    cr = cn * K_NBR
    nch = npw // cn
    assert cr <= 128 and npw % cn == 0 and nch % 2 == 0

    def reduce_chunk(buf, acc_v, c):
        for i in range(cn):
            node = c * cn + i
            for j in range(d // 16):
                acc = buf[K_NBR * i, pl.ds(16 * j, 16)]
                for k in range(1, K_NBR):
                    acc = acc + buf[K_NBR * i + k, pl.ds(16 * j, 16)]
                acc_v[node, pl.ds(16 * j, 16)] = acc

    def body(idx_hbm, tab_hbm, out_hbm, idx_v, buf0, buf1, acc_v, sem0, sem1):
        base = _wid() * rpw
        pltpu.sync_copy(idx_hbm.at[pl.ds(base, rpw)], idx_v)

        def step(g, carry):
            c0 = 2 * g
            h0 = pltpu.async_copy(
                tab_hbm.at[idx_v.at[pl.ds(c0 * cr, cr)]], buf0, sem0)
            h1 = pltpu.async_copy(
                tab_hbm.at[idx_v.at[pl.ds((c0 + 1) * cr, cr)]], buf1, sem1)
            h0.wait()
            reduce_chunk(buf0, acc_v, c0)
            h1.wait()
            reduce_chunk(buf1, acc_v, c0 + 1)
            return carry

        lax.fori_loop(0, nch // 2, step, 0)
        pltpu.sync_copy(acc_v, out_hbm.at[pl.ds(_wid() * npw, npw)])

    def run(idx, table):
        scratch = [pltpu.VMEM((rpw,), jnp.int32),
                   pltpu.VMEM((cr, d), jnp.float32),
                   pltpu.VMEM((cr, d), jnp.float32),
                   pltpu.VMEM((npw, d), jnp.float32),
                   pltpu.SemaphoreType.DMA, pltpu.SemaphoreType.DMA]
        return pl.kernel(body, mesh=_sc_mesh(),
                         out_type=jax.ShapeDtypeStruct((n_pad, d), jnp.float32),
                         scratch_types=scratch,
                         compiler_params=_SC_PARAMS)(idx, table)

    return run


def _layer1_body(a1_ref, a2_ref, nts_ref, nf1_ref, t1_ref, tw_ref, tb_ref,
                 w1a_ref, w1b_ref, w1c_ref, b1_ref,
                 w2a_ref, w2b_ref, w2c_ref, b2_ref, out_ref):
    a1 = a1_ref[...]
    nts = nts_ref[...]
    t1 = t1_ref[...]
    tw = tw_ref[...]
    tb = tb_ref[...]
    csum = jnp.zeros((a1.shape[0], D_TIME), jnp.float32)
    for k in range(K_NBR):
        csum = csum + jnp.cos((t1 - nts[:, k:k + 1]) * tw + tb)
    s = (jnp.dot(a1, w1a_ref[...], preferred_element_type=jnp.float32)
         + jnp.dot(csum, w1b_ref[...], preferred_element_type=jnp.float32)
         + jnp.dot(a2_ref[...], w1c_ref[...], preferred_element_type=jnp.float32)
         + K_NBR * b1_ref[...])
    s = jnp.maximum(s, 0.0)
    srcte = jnp.cos(tb)
    h = (jnp.dot(s, w2a_ref[...], preferred_element_type=jnp.float32)
         + jnp.dot(nf1_ref[...], w2b_ref[...], preferred_element_type=jnp.float32)
         + (jnp.dot(srcte, w2c_ref[...], preferred_element_type=jnp.float32)
            + b2_ref[...]))
    out_ref[...] = h.reshape(-1, K_NBR, D_FEAT).sum(axis=1)


def _layer2_body(b1s_ref, a2b_ref, nts_ref, nf2_ref, t2_ref, tw_ref, tb_ref,
                 w1a_ref, w1b_ref, w1c_ref, b1_ref,
                 w2a_ref, w2b_ref, w2c_ref, b2_ref,
                 fc1w_ref, fc1b_ref, fc2w_ref, fc2b_ref,
                 decw_ref, decb_ref,
                 pos_ref, neg_ref, dec_ref):
    nts = nts_ref[...]
    t2 = t2_ref[...]
    tw = tw_ref[...]
    tb = tb_ref[...]
    csum = jnp.zeros((600, D_TIME), jnp.float32)
    for k in range(K_NBR):
        csum = csum + jnp.cos((t2 - nts[:, k:k + 1]) * tw + tb)
    s = (jnp.dot(b1s_ref[...], w1a_ref[...], preferred_element_type=jnp.float32)
         + jnp.dot(csum, w1b_ref[...], preferred_element_type=jnp.float32)
         + jnp.dot(a2b_ref[...], w1c_ref[...], preferred_element_type=jnp.float32)
         + K_NBR * b1_ref[...])
    s = jnp.maximum(s, 0.0)
    srcte = jnp.cos(tb)
    emb = (jnp.dot(s, w2a_ref[...], preferred_element_type=jnp.float32)
           + jnp.dot(nf2_ref[...], w2b_ref[...], preferred_element_type=jnp.float32)
           + (jnp.dot(srcte, w2c_ref[...], preferred_element_type=jnp.float32)
              + b2_ref[...]))
    src_e = emb[:200]
    dst_e = emb[200:400]
    neg_e = emb[400:600]

    def merge(a, b):
        hh = (jnp.dot(a, fc1w_ref[:D_FEAT], preferred_element_type=jnp.float32)
              + jnp.dot(b, fc1w_ref[D_FEAT:], preferred_element_type=jnp.float32)
              + fc1b_ref[...])
        hh = jnp.maximum(hh, 0.0)
        return jnp.dot(hh, fc2w_ref[...], preferred_element_type=jnp.float32) + fc2b_ref[...]

    pos_ref[...] = jax.nn.sigmoid(merge(src_e, dst_e))
    neg_ref[...] = jax.nn.sigmoid(merge(src_e, neg_e))
    dec_ref[...] = (jnp.dot(emb[:400], decw_ref[...], preferred_element_type=jnp.float32)
                    + decb_ref[...])


def _full(a):
    return pl.BlockSpec(a.shape, lambda *_: (0,) * a.ndim)


def kernel(source_nodes, destination_nodes, negative_nodes, edge_times, edge_idxs,
           node_features, edge_features, nbr_ids, nbr_eidx, nbr_ts,
           time_w, time_b, W1, b1, W2, b2, fc1_w, fc1_b, fc2_w, fc2_b, dec_w, dec_b):
    n = source_nodes.shape[0]
    M = 3 * n
    n2 = jnp.concatenate([source_nodes, destination_nodes, negative_nodes])
    t2 = jnp.concatenate([edge_times, edge_times, edge_times])

    packed = jnp.concatenate(
        [nbr_ids, nbr_eidx, lax.bitcast_convert_type(nbr_ts, jnp.int32),
         jnp.zeros((nbr_ids.shape[0], 4), jnp.int32)], axis=1)
    n2p = jnp.pad(n2, (0, 768 - M))
    P2, NF2 = _make_row_gather(768, 24)(n2p, packed, node_features)
    NBR2 = P2[:, :K_NBR]
    EIX2 = P2[:, K_NBR:2 * K_NBR]
    NTS2 = lax.bitcast_convert_type(P2[:, 2 * K_NBR:3 * K_NBR], jnp.float32)
    n1 = NBR2[:M].reshape(-1)
    n1p = jnp.pad(n1, (0, 12288 - M * K_NBR))
    P1, NF1 = _make_row_gather(12288, 128)(n1p, packed, node_features)
    NBR1 = P1[:, :K_NBR]
    EIX1 = P1[:, K_NBR:2 * K_NBR]
    NTS1 = lax.bitcast_convert_type(P1[:, 2 * K_NBR:3 * K_NBR], jnp.float32)
    A1 = _make_gather_segsum(12288, D_FEAT, 4)(NBR1.reshape(-1), node_features)
    A2 = _make_gather_segsum(12288, D_EDGE, 4)(EIX1.reshape(-1), edge_features)
    A2b = _make_gather_segsum(768, D_EDGE, 4)(EIX2.reshape(-1), edge_features)

    tw2 = time_w.reshape(1, D_TIME)
    tb2 = time_b.reshape(1, D_TIME)
    T1 = jnp.repeat(t2, K_NBR).reshape(-1, 1)
    W1a0, W1b0, W1c0 = W1[0, :D_FEAT], W1[0, D_FEAT:D_FEAT + D_TIME], W1[0, D_FEAT + D_TIME:]
    W2a0, W2b0, W2c0 = W2[0, :D_FEAT], W2[0, D_FEAT:2 * D_FEAT], W2[0, 2 * D_FEAT:]
    W1a1, W1b1, W1c1 = W1[1, :D_FEAT], W1[1, D_FEAT:D_FEAT + D_TIME], W1[1, D_FEAT + D_TIME:]
    W2a1, W2b1, W2c1 = W2[1, :D_FEAT], W2[1, D_FEAT:2 * D_FEAT], W2[1, 2 * D_FEAT:]

    n_lvl1 = M * K_NBR
    grid = n_lvl1 // LVL1_ROWS
    row = lambda i: (i, 0)
    B1s = pl.pallas_call(
        _layer1_body,
        grid=(grid,),
        in_specs=[
            pl.BlockSpec((LVL1_ROWS, D_FEAT), row),
            pl.BlockSpec((LVL1_ROWS, D_EDGE), row),
            pl.BlockSpec((LVL1_ROWS, K_NBR), row),
            pl.BlockSpec((LVL1_ROWS, D_FEAT), row),
            pl.BlockSpec((LVL1_ROWS, 1), row),
            _full(tw2), _full(tb2),
            _full(W1a0), _full(W1b0), _full(W1c0), _full(b1[0:1]),
            _full(W2a0), _full(W2b0), _full(W2c0), _full(b2[0:1]),
        ],
        out_specs=pl.BlockSpec((LVL1_ROWS // K_NBR, D_FEAT), row),
        out_shape=jax.ShapeDtypeStruct((M, D_FEAT), jnp.float32),
    )(A1, A2, NTS1, NF1, T1, tw2, tb2,
      W1a0, W1b0, W1c0, b1[0:1], W2a0, W2b0, W2c0, b2[0:1])

    pos, neg, dec = pl.pallas_call(
        _layer2_body,
        grid=(1,),
        in_specs=[_full(B1s),
                  pl.BlockSpec((M, D_EDGE), lambda *_: (0, 0)),
                  pl.BlockSpec((M, K_NBR), lambda *_: (0, 0)),
                  pl.BlockSpec((M, D_FEAT), lambda *_: (0, 0)),
                  pl.BlockSpec((M, 1), lambda *_: (0, 0)),
                  _full(tw2), _full(tb2),
                  _full(W1a1), _full(W1b1), _full(W1c1), _full(b1[1:2]),
                  _full(W2a1), _full(W2b1), _full(W2c1), _full(b2[1:2]),
                  _full(fc1_w), _full(fc1_b.reshape(1, D_FEAT)),
                  _full(fc2_w), _full(fc2_b.reshape(1, 1)),
                  _full(dec_w), _full(dec_b.reshape(1, D_FEAT))],
        out_specs=[pl.BlockSpec((n, 1), lambda *_: (0, 0)),
                   pl.BlockSpec((n, 1), lambda *_: (0, 0)),
                   pl.BlockSpec((2 * n, D_FEAT), lambda *_: (0, 0))],
        out_shape=[jax.ShapeDtypeStruct((n, 1), jnp.float32),
                   jax.ShapeDtypeStruct((n, 1), jnp.float32),
                   jax.ShapeDtypeStruct((2 * n, D_FEAT), jnp.float32)],
    )(B1s, A2b, NTS2, NF2, t2.reshape(M, 1), tw2, tb2,
      W1a1, W1b1, W1c1, b1[1:2], W2a1, W2b1, W2c1, b2[1:2],
      fc1_w, fc1_b.reshape(1, D_FEAT), fc2_w, fc2_b.reshape(1, 1),
      dec_w, dec_b.reshape(1, D_FEAT))

    return pos.reshape(-1), neg.reshape(-1), dec, NF2[:2 * n]
